# baseline (speedup 1.0000x reference)
import functools

import jax
import jax.numpy as jnp
from jax.experimental import pallas as pl
from jax.experimental.pallas import tpu as pltpu
from jax.experimental.pallas import tpu_sc as plsc

G = 4


def _src_tap(s, i, j):
    if s == 0:
        return (i, j)
    if s == 1:
        return (j, 2 - i)
    if s == 2:
        return (2 - i, 2 - j)
    return (2 - j, i)


def _sc_transform(kernel, bias):
    kh, kw, gcin, cout = kernel.shape
    cin = gcin // G
    n_out = G * cout
    mesh = plsc.VectorSubcoreMesh(core_axis_name="c", subcore_axis_name="s")

    @functools.partial(
        pl.kernel,
        out_type=(
            jax.ShapeDtypeStruct((G, kh, kw * gcin, cout), jnp.float32),
            jax.ShapeDtypeStruct((G, 1, cout), jnp.float32),
        ),
        mesh=mesh,
    )
    def tbody(k_hbm, b_hbm, wt_hbm, rb_hbm):
        wid = jax.lax.axis_index("s") * 2 + jax.lax.axis_index("c")

        def weight_task(i, j, s):
            si, sj = _src_tap(s, i, j)
            sh = s * cin
            base = j * gcin
            if sh:
                pltpu.sync_copy(
                    k_hbm.at[si, sj, pl.ds(gcin - sh, sh), :],
                    wt_hbm.at[s, i, pl.ds(base, sh), :])
                pltpu.sync_copy(
                    k_hbm.at[si, sj, pl.ds(0, gcin - sh), :],
                    wt_hbm.at[s, i, pl.ds(base + sh, gcin - sh), :])
            else:
                pltpu.sync_copy(
                    k_hbm.at[si, sj, pl.ds(0, gcin), :],
                    wt_hbm.at[s, i, pl.ds(base, gcin), :])

        t = 0
        for i in range(3):
            for j in range(3):
                for s in range(G):
                    pl.when(wid == (t % 32))(
                        functools.partial(weight_task, i, j, s))
                    t += 1

        def bias_task(s):
            pltpu.sync_copy(b_hbm.at[:, :], rb_hbm.at[s])

        for s in range(G):
            pl.when(wid == ((t + s) % 32))(functools.partial(bias_task, s))

    return tbody(kernel, bias.reshape(1, cout))


def _conv_body(x_hbm, wt_ref, rb_ref, out_ref, buf, wtc, sems, *,
               th, w, n_tiles):
    r = pl.program_id(0)
    kdim = x_hbm.shape[2]
    n = wt_ref.shape[0] * wt_ref.shape[3]
    hw = th + 2

    def issue(t):
        slot = jax.lax.rem(t, 2)

        @pl.when(t == 0)
        def _():
            buf[slot, 0] = jnp.zeros((w, kdim), buf.dtype)
            pltpu.make_async_copy(
                x_hbm.at[pl.ds(0, th + 1)],
                buf.at[slot, pl.ds(1, th + 1)],
                sems.at[slot]).start()

        @pl.when(jnp.logical_and(t > 0, t < n_tiles - 1))
        def _():
            pltpu.make_async_copy(
                x_hbm.at[pl.ds(jnp.clip(t * th - 1, 0, x_hbm.shape[0] - hw),
                               hw)],
                buf.at[slot, pl.ds(0, hw)],
                sems.at[slot]).start()

        @pl.when(t == n_tiles - 1)
        def _():
            buf[slot, hw - 1] = jnp.zeros((w, kdim), buf.dtype)
            pltpu.make_async_copy(
                x_hbm.at[pl.ds((n_tiles - 1) * th - 1, th + 1)],
                buf.at[slot, pl.ds(0, th + 1)],
                sems.at[slot]).start()

    def wait(t):
        slot = jax.lax.rem(t, 2)
        edge = jnp.logical_or(t == 0, t == n_tiles - 1)

        @pl.when(edge)
        def _():
            pltpu.make_async_copy(
                x_hbm.at[pl.ds(0, th + 1)],
                buf.at[slot, pl.ds(1, th + 1)],
                sems.at[slot]).wait()

        @pl.when(jnp.logical_not(edge))
        def _():
            pltpu.make_async_copy(
                x_hbm.at[pl.ds(0, hw)],
                buf.at[slot, pl.ds(0, hw)],
                sems.at[slot]).wait()

    @pl.when(r == 0)
    def _():
        for i in range(3):
            wtc[i] = jnp.concatenate(
                [wt_ref[s, i] for s in range(wt_ref.shape[0])],
                axis=1).astype(jnp.bfloat16)
        issue(0)

    @pl.when(r + 1 < n_tiles)
    def _():
        issue(r + 1)

    wait(r)

    slot = jax.lax.rem(r, 2)
    xc = buf[slot].astype(jnp.bfloat16)
    zcol = jnp.zeros((hw, 1, kdim), jnp.bfloat16)
    bufw = jnp.concatenate([
        jnp.concatenate([zcol, xc[:, :w - 1]], axis=1),
        xc,
        jnp.concatenate([xc[:, 1:], zcol], axis=1),
    ], axis=2)
    acc = jnp.dot(bufw[0:th].reshape(th * w, 3 * kdim), wtc[0],
                  preferred_element_type=jnp.float32)
    for i in range(1, 3):
        acc += jnp.dot(bufw[i:i + th].reshape(th * w, 3 * kdim), wtc[i],
                       preferred_element_type=jnp.float32)
    rb_row = jnp.concatenate([rb_ref[s] for s in range(rb_ref.shape[0])],
                             axis=1)
    out_ref[...] = (acc + rb_row).reshape(th, w, n)


def kernel(inputs, kernel, bias):
    B, H, W, G_, C = inputs.shape
    kh, kw, gcin, cout = kernel.shape
    n_out = G * cout

    x = inputs.reshape(H, W, G_ * C)

    wt, rb = _sc_transform(kernel, bias)

    TH = 16
    n_tiles = H // TH

    conv = pl.pallas_call(
        lambda x_hbm, wt_ref, rb_ref, out_ref, buf, wtc, sems: _conv_body(
            x_hbm, wt_ref, rb_ref, out_ref, buf, wtc, sems,
            th=TH, w=W, n_tiles=n_tiles),
        grid=(n_tiles,),
        in_specs=[
            pl.BlockSpec(memory_space=pltpu.MemorySpace.HBM),
            pl.BlockSpec((G, kh, kw * gcin, cout), lambda r: (0, 0, 0, 0)),
            pl.BlockSpec((G, 1, cout), lambda r: (0, 0, 0)),
        ],
        out_specs=pl.BlockSpec((TH, W, n_out), lambda r: (r, 0, 0)),
        out_shape=jax.ShapeDtypeStruct((H, W, n_out), jnp.float32),
        scratch_shapes=[
            pltpu.VMEM((2, TH + 2, W, gcin), jnp.float32),
            pltpu.VMEM((kh, kw * gcin, n_out), jnp.bfloat16),
            pltpu.SemaphoreType.DMA((2,)),
        ],
        compiler_params=pltpu.CompilerParams(
            dimension_semantics=("parallel",),
        ),
    )(x, wt, rb)

    return conv.reshape(B, H, W, G, cout)


# speedup vs baseline: 1.3734x; 1.3734x over previous
import functools

import jax
import jax.numpy as jnp
from jax.experimental import pallas as pl
from jax.experimental.pallas import tpu as pltpu
from jax.experimental.pallas import tpu_sc as plsc

G = 4


def _src_tap(s, i, j):
    if s == 0:
        return (i, j)
    if s == 1:
        return (j, 2 - i)
    if s == 2:
        return (2 - i, 2 - j)
    return (2 - j, i)


def _sc_transform(kernel, bias):
    kh, kw, gcin, cout = kernel.shape
    cin = gcin // G
    n_out = G * cout
    mesh = plsc.VectorSubcoreMesh(core_axis_name="c", subcore_axis_name="s")

    @functools.partial(
        pl.kernel,
        out_type=(
            jax.ShapeDtypeStruct((G, kh, kw * gcin, cout), jnp.float32),
            jax.ShapeDtypeStruct((G, 1, cout), jnp.float32),
        ),
        scratch_types=[pltpu.VMEM((gcin, cout), jnp.float32)],
        mesh=mesh,
    )
    def tbody(k_hbm, b_hbm, wt_hbm, rb_hbm, vbuf):
        wid = jax.lax.axis_index("s") * 2 + jax.lax.axis_index("c")

        def weight_task(i, j, s):
            si, sj = _src_tap(s, i, j)
            sh = s * cin
            base = j * gcin
            pltpu.sync_copy(k_hbm.at[si, sj], vbuf)
            if sh:
                pltpu.sync_copy(vbuf.at[pl.ds(gcin - sh, sh), :],
                                wt_hbm.at[s, i, pl.ds(base, sh), :])
                pltpu.sync_copy(vbuf.at[pl.ds(0, gcin - sh), :],
                                wt_hbm.at[s, i, pl.ds(base + sh, gcin - sh), :])
            else:
                pltpu.sync_copy(vbuf,
                                wt_hbm.at[s, i, pl.ds(base, gcin), :])

        t = 0
        for i in range(3):
            for j in range(3):
                for s in range(G):
                    pl.when(wid == (t % 32))(
                        functools.partial(weight_task, i, j, s))
                    t += 1

        def bias_task(s):
            pltpu.sync_copy(b_hbm.at[:, :], vbuf.at[pl.ds(0, 1), :])
            pltpu.sync_copy(vbuf.at[pl.ds(0, 1), :], rb_hbm.at[s])

        for s in range(G):
            pl.when(wid == ((t + s) % 32))(functools.partial(bias_task, s))

    return tbody(kernel, bias.reshape(1, cout))


def _conv_body(x_hbm, wt_ref, rb_ref, out_ref, buf, wtc, sems, *,
               th, w, n_tiles):
    r = pl.program_id(0)
    kdim = x_hbm.shape[2]
    n = wt_ref.shape[0] * wt_ref.shape[3]
    hw = th + 2

    def issue(t):
        slot = jax.lax.rem(t, 2)

        @pl.when(t == 0)
        def _():
            buf[slot, 0] = jnp.zeros((w, kdim), buf.dtype)
            pltpu.make_async_copy(
                x_hbm.at[pl.ds(0, th + 1)],
                buf.at[slot, pl.ds(1, th + 1)],
                sems.at[slot]).start()

        @pl.when(jnp.logical_and(t > 0, t < n_tiles - 1))
        def _():
            pltpu.make_async_copy(
                x_hbm.at[pl.ds(jnp.clip(t * th - 1, 0, x_hbm.shape[0] - hw),
                               hw)],
                buf.at[slot, pl.ds(0, hw)],
                sems.at[slot]).start()

        @pl.when(t == n_tiles - 1)
        def _():
            buf[slot, hw - 1] = jnp.zeros((w, kdim), buf.dtype)
            pltpu.make_async_copy(
                x_hbm.at[pl.ds((n_tiles - 1) * th - 1, th + 1)],
                buf.at[slot, pl.ds(0, th + 1)],
                sems.at[slot]).start()

    def wait(t):
        slot = jax.lax.rem(t, 2)
        edge = jnp.logical_or(t == 0, t == n_tiles - 1)

        @pl.when(edge)
        def _():
            pltpu.make_async_copy(
                x_hbm.at[pl.ds(0, th + 1)],
                buf.at[slot, pl.ds(1, th + 1)],
                sems.at[slot]).wait()

        @pl.when(jnp.logical_not(edge))
        def _():
            pltpu.make_async_copy(
                x_hbm.at[pl.ds(0, hw)],
                buf.at[slot, pl.ds(0, hw)],
                sems.at[slot]).wait()

    @pl.when(r == 0)
    def _():
        for i in range(3):
            wtc[i] = jnp.concatenate(
                [wt_ref[s, i] for s in range(wt_ref.shape[0])],
                axis=1).astype(jnp.bfloat16)
        issue(0)

    @pl.when(r + 1 < n_tiles)
    def _():
        issue(r + 1)

    wait(r)

    slot = jax.lax.rem(r, 2)
    xc = buf[slot].astype(jnp.bfloat16)
    zcol = jnp.zeros((hw, 1, kdim), jnp.bfloat16)
    bufw = jnp.concatenate([
        jnp.concatenate([zcol, xc[:, :w - 1]], axis=1),
        xc,
        jnp.concatenate([xc[:, 1:], zcol], axis=1),
    ], axis=2)
    acc = jnp.dot(bufw[0:th].reshape(th * w, 3 * kdim), wtc[0],
                  preferred_element_type=jnp.float32)
    for i in range(1, 3):
        acc += jnp.dot(bufw[i:i + th].reshape(th * w, 3 * kdim), wtc[i],
                       preferred_element_type=jnp.float32)
    rb_row = jnp.concatenate([rb_ref[s] for s in range(rb_ref.shape[0])],
                             axis=1)
    out_ref[...] = (acc + rb_row).reshape(th, w, n)


def kernel(inputs, kernel, bias):
    B, H, W, G_, C = inputs.shape
    kh, kw, gcin, cout = kernel.shape
    n_out = G * cout

    x = inputs.reshape(H, W, G_ * C)

    wt, rb = _sc_transform(kernel, bias)

    TH = 16
    n_tiles = H // TH

    conv = pl.pallas_call(
        lambda x_hbm, wt_ref, rb_ref, out_ref, buf, wtc, sems: _conv_body(
            x_hbm, wt_ref, rb_ref, out_ref, buf, wtc, sems,
            th=TH, w=W, n_tiles=n_tiles),
        grid=(n_tiles,),
        in_specs=[
            pl.BlockSpec(memory_space=pltpu.MemorySpace.HBM),
            pl.BlockSpec((G, kh, kw * gcin, cout), lambda r: (0, 0, 0, 0)),
            pl.BlockSpec((G, 1, cout), lambda r: (0, 0, 0)),
        ],
        out_specs=pl.BlockSpec((TH, W, n_out), lambda r: (r, 0, 0)),
        out_shape=jax.ShapeDtypeStruct((H, W, n_out), jnp.float32),
        scratch_shapes=[
            pltpu.VMEM((2, TH + 2, W, gcin), jnp.float32),
            pltpu.VMEM((kh, kw * gcin, n_out), jnp.bfloat16),
            pltpu.SemaphoreType.DMA((2,)),
        ],
        compiler_params=pltpu.CompilerParams(
            dimension_semantics=("parallel",),
        ),
    )(x, wt, rb)

    return conv.reshape(B, H, W, G, cout)
